# baseline (speedup 1.0000x reference)
import functools

import jax
import jax.numpy as jnp
from jax.experimental import pallas as pl
from jax.experimental.pallas import tpu as pltpu

_TEMP = 0.1
_EPS = 1e-5
_TOPK = 8


def _step(x_ref, w1_ref, gamma_ref, beta_ref, w2_ref, b2_ref,
          rw_ref, idx_ref, logits_ref,
          cast_dst, dot_src, dot_acc, epi_src, st_ref,
          *, j, n_j, tn, n_experts, dcol, scol):
    h_dim = dot_src.shape[1]
    sl = pl.ds(j * tn, tn)
    zero = (j == 0)

    cast_dst[:, sl] = x_ref[...].astype(jnp.bfloat16)

    part = jax.lax.dot_general(
        dot_src[...], w1_ref[...],
        (((1,), (1,)), ((), ())),
        preferred_element_type=jnp.float32,
    )
    dot_acc[:, sl] = part
    cs = jnp.sum(part, axis=1, keepdims=True)
    cq = jnp.sum(part * part, axis=1, keepdims=True)
    st_ref[:, dcol:dcol + 1] = jnp.where(zero, 0.0, st_ref[:, dcol:dcol + 1]) + cs
    st_ref[:, dcol + 1:dcol + 2] = (
        jnp.where(zero, 0.0, st_ref[:, dcol + 1:dcol + 2]) + cq)

    mu = st_ref[:, scol:scol + 1] / h_dim
    var = st_ref[:, scol + 1:scol + 2] / h_dim - mu * mu
    rs = jax.lax.rsqrt(var + _EPS)
    hc = epi_src[:, sl]
    ln = (hc - mu) * rs * gamma_ref[:, sl] + beta_ref[:, sl]
    s = ln * jax.nn.sigmoid(ln)
    pl2 = jax.lax.dot_general(
        s.astype(jnp.bfloat16), w2_ref[:, sl],
        (((1,), (1,)), ((), ())),
        preferred_element_type=jnp.float32,
    )
    logits_ref[...] = jnp.where(zero, b2_ref[...], logits_ref[...]) + pl2

    @pl.when(j == n_j - 1)
    def _finish():
        logits = logits_ref[...]
        z = logits / _TEMP
        z = z - jnp.max(z, axis=1, keepdims=True)
        ez = jnp.exp(z)
        w = ez / jnp.sum(ez, axis=1, keepdims=True)

        tm = w.shape[0]
        ii = jax.lax.broadcasted_iota(jnp.int32, (tm, n_experts), 1)
        cur = w
        vals, idxs = [], []
        for _ in range(_TOPK):
            m = jnp.max(cur, axis=1, keepdims=True)
            jj = jnp.min(jnp.where(cur == m, ii, n_experts), axis=1, keepdims=True)
            vals.append(m)
            idxs.append(jj)
            cur = jnp.where(ii == jj, -1.0, cur)
        rw_ref[...] = jnp.concatenate(vals, axis=1)
        idx_ref[...] = jnp.concatenate(idxs, axis=1)


def _router_kernel(x_ref, w1_ref, gamma_ref, beta_ref, w2_ref, b2_ref,
                   rw_ref, idx_ref, logits_ref,
                   acc_a, acc_b, xbf_a, xbf_b, st_ref,
                   *, n_j, tn, n_experts):
    s = pl.program_id(0)
    j = pl.program_id(1)
    outs = (rw_ref, idx_ref, logits_ref)
    ins = (x_ref, w1_ref, gamma_ref, beta_ref, w2_ref, b2_ref)
    step = functools.partial(_step, j=j, n_j=n_j, tn=tn, n_experts=n_experts)

    @pl.when(jax.lax.rem(s, 2) == 0)
    def _even():
        step(*ins, *outs, xbf_a, xbf_b, acc_b, acc_a, st_ref, dcol=2, scol=0)

    @pl.when(jax.lax.rem(s, 2) == 1)
    def _odd():
        step(*ins, *outs, xbf_b, xbf_a, acc_a, acc_b, st_ref, dcol=0, scol=2)


def kernel(x, W1, gamma, beta, W2, b2):
    n_tok, h_dim = x.shape
    n_experts = W2.shape[0]
    tm = min(512, n_tok)
    tn = min(1024, h_dim)
    n_i = n_tok // tm
    n_j = h_dim // tn

    w1_bf = W1.astype(jnp.bfloat16)
    w2_bf = W2.astype(jnp.bfloat16)
    gamma2 = gamma.reshape(1, h_dim)
    beta2 = beta.reshape(1, h_dim)
    b22 = b2.reshape(1, n_experts)

    last = n_i - 1
    body = functools.partial(_router_kernel, n_j=n_j, tn=tn, n_experts=n_experts)
    rw, idx, logits = pl.pallas_call(
        body,
        grid=(n_i + 2, n_j),
        in_specs=[
            pl.BlockSpec((tm, tn), lambda s, j: (jnp.minimum(s, last), j)),
            pl.BlockSpec((tn, h_dim), lambda s, j: (j, 0)),
            pl.BlockSpec((1, h_dim), lambda s, j: (0, 0)),
            pl.BlockSpec((1, h_dim), lambda s, j: (0, 0)),
            pl.BlockSpec((n_experts, h_dim), lambda s, j: (0, 0)),
            pl.BlockSpec((1, n_experts), lambda s, j: (0, 0)),
        ],
        out_specs=[
            pl.BlockSpec((tm, _TOPK), lambda s, j: (jnp.maximum(s - 2, 0), 0)),
            pl.BlockSpec((tm, _TOPK), lambda s, j: (jnp.maximum(s - 2, 0), 0)),
            pl.BlockSpec((tm, n_experts), lambda s, j: (jnp.maximum(s - 2, 0), 0)),
        ],
        out_shape=[
            jax.ShapeDtypeStruct((n_tok, _TOPK), jnp.float32),
            jax.ShapeDtypeStruct((n_tok, _TOPK), jnp.int32),
            jax.ShapeDtypeStruct((n_tok, n_experts), jnp.float32),
        ],
        scratch_shapes=[
            pltpu.VMEM((tm, h_dim), jnp.float32),
            pltpu.VMEM((tm, h_dim), jnp.float32),
            pltpu.VMEM((tm, h_dim), jnp.bfloat16),
            pltpu.VMEM((tm, h_dim), jnp.bfloat16),
            pltpu.VMEM((tm, 4), jnp.float32),
        ],
        compiler_params=pltpu.CompilerParams(
            dimension_semantics=("arbitrary", "arbitrary"),
        ),
    )(x, w1_bf, gamma2, beta2, w2_bf, b22)
    return (rw, idx, logits)


# speedup vs baseline: 1.1573x; 1.1573x over previous
import functools

import jax
import jax.numpy as jnp
from jax.experimental import pallas as pl
from jax.experimental.pallas import tpu as pltpu

_TEMP = 0.1
_EPS = 1e-5
_TOPK = 8


def _step(x_ref, w1_ref, gamma_ref, beta_ref, w2_ref, b2_ref,
          rw_ref, idx_ref, logits_ref,
          dst_ref, dsum_ref, dsq_ref,
          src_ref, ssum_ref, ssq_ref,
          lg_ref, *, j, n_j, tn, n_experts):
    h_dim = src_ref.shape[1]
    sl = pl.ds(j * tn, tn)

    part = jax.lax.dot_general(
        x_ref[...].astype(jnp.bfloat16), w1_ref[...],
        (((1,), (1,)), ((), ())),
        preferred_element_type=jnp.float32,
    )
    dst_ref[:, sl] = part
    cs = jnp.sum(part, axis=1, keepdims=True)
    cq = jnp.sum(part * part, axis=1, keepdims=True)
    zero = (j == 0)
    dsum_ref[...] = jnp.where(zero, 0.0, dsum_ref[...]) + cs
    dsq_ref[...] = jnp.where(zero, 0.0, dsq_ref[...]) + cq

    mu = ssum_ref[...] / h_dim
    var = ssq_ref[...] / h_dim - mu * mu
    rs = jax.lax.rsqrt(var + _EPS)
    hc = src_ref[:, sl]
    ln = (hc - mu) * rs * gamma_ref[:, sl] + beta_ref[:, sl]
    s = ln * jax.nn.sigmoid(ln)
    pl2 = jax.lax.dot_general(
        s.astype(jnp.bfloat16), w2_ref[:, sl],
        (((1,), (1,)), ((), ())),
        preferred_element_type=jnp.float32,
    )
    lg_ref[...] = jnp.where(zero, b2_ref[...], lg_ref[...]) + pl2

    @pl.when(j == n_j - 1)
    def _finish():
        logits = lg_ref[...]
        logits_ref[...] = logits
        z = logits / _TEMP
        z = z - jnp.max(z, axis=1, keepdims=True)
        ez = jnp.exp(z)
        w = ez / jnp.sum(ez, axis=1, keepdims=True)

        tm = w.shape[0]
        ii = jax.lax.broadcasted_iota(jnp.int32, (tm, n_experts), 1)
        cur = w
        vals, idxs = [], []
        for _ in range(_TOPK):
            m = jnp.max(cur, axis=1, keepdims=True)
            jj = jnp.min(jnp.where(cur == m, ii, n_experts), axis=1, keepdims=True)
            vals.append(m)
            idxs.append(jj)
            cur = jnp.where(ii == jj, -1.0, cur)
        rw_ref[...] = jnp.concatenate(vals, axis=1)
        idx_ref[...] = jnp.concatenate(idxs, axis=1)


def _router_kernel(x_ref, w1_ref, gamma_ref, beta_ref, w2_ref, b2_ref,
                   rw_ref, idx_ref, logits_ref,
                   acc_a, acc_b, sum_a, sq_a, sum_b, sq_b, lg_ref,
                   *, n_j, tn, n_experts):
    i = pl.program_id(0)
    j = pl.program_id(1)
    outs = (rw_ref, idx_ref, logits_ref)
    ins = (x_ref, w1_ref, gamma_ref, beta_ref, w2_ref, b2_ref)
    step = functools.partial(_step, j=j, n_j=n_j, tn=tn, n_experts=n_experts)

    @pl.when(jax.lax.rem(i, 2) == 0)
    def _even():
        step(*ins, *outs, acc_a, sum_a, sq_a, acc_b, sum_b, sq_b, lg_ref)

    @pl.when(jax.lax.rem(i, 2) == 1)
    def _odd():
        step(*ins, *outs, acc_b, sum_b, sq_b, acc_a, sum_a, sq_a, lg_ref)


def kernel(x, W1, gamma, beta, W2, b2):
    n_tok, h_dim = x.shape
    n_experts = W2.shape[0]
    tm = min(512, n_tok)
    tn = min(1024, h_dim)
    n_i = n_tok // tm
    n_j = h_dim // tn

    w1_bf = W1.astype(jnp.bfloat16)
    w2_bf = W2.astype(jnp.bfloat16)
    gamma2 = gamma.reshape(1, h_dim)
    beta2 = beta.reshape(1, h_dim)
    b22 = b2.reshape(1, n_experts)

    last = n_i - 1
    body = functools.partial(_router_kernel, n_j=n_j, tn=tn, n_experts=n_experts)
    rw, idx, logits = pl.pallas_call(
        body,
        grid=(n_i + 1, n_j),
        in_specs=[
            pl.BlockSpec((tm, h_dim), lambda i, j: (jnp.minimum(i, last), 0)),
            pl.BlockSpec((tn, h_dim), lambda i, j: (j, 0)),
            pl.BlockSpec((1, h_dim), lambda i, j: (0, 0)),
            pl.BlockSpec((1, h_dim), lambda i, j: (0, 0)),
            pl.BlockSpec((n_experts, h_dim), lambda i, j: (0, 0)),
            pl.BlockSpec((1, n_experts), lambda i, j: (0, 0)),
        ],
        out_specs=[
            pl.BlockSpec((tm, _TOPK), lambda i, j: (jnp.maximum(i - 1, 0), 0)),
            pl.BlockSpec((tm, _TOPK), lambda i, j: (jnp.maximum(i - 1, 0), 0)),
            pl.BlockSpec((tm, n_experts), lambda i, j: (jnp.maximum(i - 1, 0), 0)),
        ],
        out_shape=[
            jax.ShapeDtypeStruct((n_tok, _TOPK), jnp.float32),
            jax.ShapeDtypeStruct((n_tok, _TOPK), jnp.int32),
            jax.ShapeDtypeStruct((n_tok, n_experts), jnp.float32),
        ],
        scratch_shapes=[
            pltpu.VMEM((tm, h_dim), jnp.float32),
            pltpu.VMEM((tm, h_dim), jnp.float32),
            pltpu.VMEM((tm, 1), jnp.float32),
            pltpu.VMEM((tm, 1), jnp.float32),
            pltpu.VMEM((tm, 1), jnp.float32),
            pltpu.VMEM((tm, 1), jnp.float32),
            pltpu.VMEM((tm, n_experts), jnp.float32),
        ],
        compiler_params=pltpu.CompilerParams(
            dimension_semantics=("arbitrary", "arbitrary"),
        ),
    )(x, w1_bf, gamma2, beta2, w2_bf, b22)
    return (rw, idx, logits)


# speedup vs baseline: 1.1585x; 1.0011x over previous
import functools

import jax
import jax.numpy as jnp
from jax.experimental import pallas as pl
from jax.experimental.pallas import tpu as pltpu

_TEMP = 0.1
_EPS = 1e-5
_TOPK = 8


def _step(x_ref, w1_ref, gamma_ref, beta_ref, w2_ref, b2_ref,
          rw_ref, idx_ref, logits_ref,
          dst_ref, dsum_ref, dsq_ref,
          src_ref, ssum_ref, ssq_ref,
          lg_ref, *, j, n_j, tn, n_experts):
    h_dim = src_ref.shape[1]
    sl = pl.ds(j * tn, tn)

    part = jax.lax.dot_general(
        x_ref[...].astype(jnp.bfloat16), w1_ref[...],
        (((1,), (1,)), ((), ())),
        preferred_element_type=jnp.float32,
    )
    dst_ref[:, sl] = part
    cs = jnp.sum(part, axis=1, keepdims=True)
    cq = jnp.sum(part * part, axis=1, keepdims=True)
    zero = (j == 0)
    dsum_ref[...] = jnp.where(zero, 0.0, dsum_ref[...]) + cs
    dsq_ref[...] = jnp.where(zero, 0.0, dsq_ref[...]) + cq

    mu = ssum_ref[...] / h_dim
    var = ssq_ref[...] / h_dim - mu * mu
    sd = jnp.sqrt(var + _EPS)
    hc = src_ref[:, sl]
    ln = (hc - mu) / sd * gamma_ref[:, sl] + beta_ref[:, sl]
    s = ln * jax.nn.sigmoid(ln)
    pl2 = jax.lax.dot_general(
        s.astype(jnp.bfloat16), w2_ref[:, sl],
        (((1,), (1,)), ((), ())),
        preferred_element_type=jnp.float32,
    )
    lg_ref[...] = jnp.where(zero, b2_ref[...], lg_ref[...]) + pl2

    @pl.when(j == n_j - 1)
    def _finish():
        logits = lg_ref[...]
        logits_ref[...] = logits
        z = logits / _TEMP
        z = z - jnp.max(z, axis=1, keepdims=True)
        ez = jnp.exp(z)
        w = ez / jnp.sum(ez, axis=1, keepdims=True)

        tm = w.shape[0]
        ii = jax.lax.broadcasted_iota(jnp.int32, (tm, n_experts), 1)
        cur = w
        vals, idxs = [], []
        for _ in range(_TOPK):
            m = jnp.max(cur, axis=1, keepdims=True)
            jj = jnp.min(jnp.where(cur == m, ii, n_experts), axis=1, keepdims=True)
            vals.append(m)
            idxs.append(jj)
            cur = jnp.where(ii == jj, -1.0, cur)
        rw_ref[...] = jnp.concatenate(vals, axis=1)
        idx_ref[...] = jnp.concatenate(idxs, axis=1)


def _router_kernel(x_ref, w1_ref, gamma_ref, beta_ref, w2_ref, b2_ref,
                   rw_ref, idx_ref, logits_ref,
                   acc_a, acc_b, sum_a, sq_a, sum_b, sq_b, lg_ref,
                   *, n_j, tn, n_experts):
    i = pl.program_id(0)
    j = pl.program_id(1)
    outs = (rw_ref, idx_ref, logits_ref)
    ins = (x_ref, w1_ref, gamma_ref, beta_ref, w2_ref, b2_ref)
    step = functools.partial(_step, j=j, n_j=n_j, tn=tn, n_experts=n_experts)

    @pl.when(jax.lax.rem(i, 2) == 0)
    def _even():
        step(*ins, *outs, acc_a, sum_a, sq_a, acc_b, sum_b, sq_b, lg_ref)

    @pl.when(jax.lax.rem(i, 2) == 1)
    def _odd():
        step(*ins, *outs, acc_b, sum_b, sq_b, acc_a, sum_a, sq_a, lg_ref)


def kernel(x, W1, gamma, beta, W2, b2):
    n_tok, h_dim = x.shape
    n_experts = W2.shape[0]
    tm = min(512, n_tok)
    tn = min(1024, h_dim)
    n_i = n_tok // tm
    n_j = h_dim // tn

    w1_bf = W1.astype(jnp.bfloat16)
    w2_bf = W2.astype(jnp.bfloat16)
    gamma2 = gamma.reshape(1, h_dim)
    beta2 = beta.reshape(1, h_dim)
    b22 = b2.reshape(1, n_experts)

    last = n_i - 1
    body = functools.partial(_router_kernel, n_j=n_j, tn=tn, n_experts=n_experts)
    rw, idx, logits = pl.pallas_call(
        body,
        grid=(n_i + 1, n_j),
        in_specs=[
            pl.BlockSpec((tm, h_dim), lambda i, j: (jnp.minimum(i, last), 0)),
            pl.BlockSpec((tn, h_dim), lambda i, j: (j, 0)),
            pl.BlockSpec((1, h_dim), lambda i, j: (0, 0)),
            pl.BlockSpec((1, h_dim), lambda i, j: (0, 0)),
            pl.BlockSpec((n_experts, h_dim), lambda i, j: (0, 0)),
            pl.BlockSpec((1, n_experts), lambda i, j: (0, 0)),
        ],
        out_specs=[
            pl.BlockSpec((tm, _TOPK), lambda i, j: (jnp.maximum(i - 1, 0), 0)),
            pl.BlockSpec((tm, _TOPK), lambda i, j: (jnp.maximum(i - 1, 0), 0)),
            pl.BlockSpec((tm, n_experts), lambda i, j: (jnp.maximum(i - 1, 0), 0)),
        ],
        out_shape=[
            jax.ShapeDtypeStruct((n_tok, _TOPK), jnp.float32),
            jax.ShapeDtypeStruct((n_tok, _TOPK), jnp.int32),
            jax.ShapeDtypeStruct((n_tok, n_experts), jnp.float32),
        ],
        scratch_shapes=[
            pltpu.VMEM((tm, h_dim), jnp.float32),
            pltpu.VMEM((tm, h_dim), jnp.float32),
            pltpu.VMEM((tm, 1), jnp.float32),
            pltpu.VMEM((tm, 1), jnp.float32),
            pltpu.VMEM((tm, 1), jnp.float32),
            pltpu.VMEM((tm, 1), jnp.float32),
            pltpu.VMEM((tm, n_experts), jnp.float32),
        ],
        compiler_params=pltpu.CompilerParams(
            dimension_semantics=("arbitrary", "arbitrary"),
        ),
    )(x, w1_bf, gamma2, beta2, w2_bf, b22)
    return (rw, idx, logits)
